# baseline (speedup 1.0000x reference)
import functools

import jax
import jax.numpy as jnp
from jax import lax
from jax.experimental import pallas as pl
from jax.experimental.pallas import tpu as pltpu
from jax.experimental.pallas import tpu_sc as plsc

_NC, _NS = 2, 16
_NW = _NC * _NS
_LANE = 16
_C = 8
_NR = 3


def _build(B, T, V, D):
    C = _C
    NR = _NR
    PT = T // _NW
    NG = PT // C
    lanes_per_row = D // _LANE

    mesh = plsc.VectorSubcoreMesh(
        core_axis_name="c", subcore_axis_name="s",
        num_cores=_NC, num_subcores=_NS)

    @functools.partial(
        pl.kernel,
        mesh=mesh,
        out_type=jax.ShapeDtypeStruct((B * T, D), jnp.float32),
        scratch_types=[
            pltpu.VMEM((B * PT,), jnp.int32),
            pltpu.VMEM((NR, B, C, D), jnp.float32),
            pltpu.VMEM((NR, C, D), jnp.float32),
            pltpu.SemaphoreType.DMA,
            [[pltpu.SemaphoreType.DMA] * B] * _NR,
            [[pltpu.SemaphoreType.DMA] * B] * _NR,
            [pltpu.SemaphoreType.DMA] * _NR,
        ],
    )
    def k(idx_hbm, tok_hbm, pos_hbm, out_hbm, idx_v, tok_v, pos_v,
          isem, gsems, osems, psems):
        wid = lax.axis_index("s") * _NC + lax.axis_index("c")
        t0 = wid * PT

        def pos_cp(g, r):
            return pltpu.make_async_copy(
                pos_hbm.at[pl.ds(t0 + g * C, C)], pos_v.at[r], psems[r])

        def gather_cp(g, r, b):
            return pltpu.make_async_copy(
                tok_hbm.at[idx_v.at[pl.ds(b * PT + g * C, C)]],
                tok_v.at[r, b], gsems[r][b])

        def out_cp(g, r, b):
            return pltpu.make_async_copy(
                tok_v.at[r, b], out_hbm.at[pl.ds(b * T + t0 + g * C, C)],
                osems[r][b])

        def fused_add(r):
            def row_body(row, _):
                for j in range(lanes_per_row):
                    s = pl.ds(j * _LANE, _LANE)
                    p = pos_v[r, row, s]
                    for b in range(B):
                        plsc.addupdate(tok_v.at[r, b, row, s], p)
                return 0
            lax.fori_loop(0, C, row_body, 0)

        def run_group(g, gr, first=False, prefetch=True, last=False):
            nr = (gr + 1) % NR
            pos_cp(g, gr).wait()
            if prefetch:
                pos_cp(g + 2, (gr + 2) % NR).start()
            if not first:
                for b in range(B):
                    out_cp(g - 2, nr, b).wait()
            if not last:
                for b in range(B):
                    gather_cp(g + 1, nr, b).start()
            for b in range(B):
                gather_cp(g, gr, b).wait()
            fused_add(gr)
            for b in range(B):
                out_cp(g, gr, b).start()

        idx_cps = [
            pltpu.async_copy(idx_hbm.at[b, pl.ds(t0, PT)],
                             idx_v.at[pl.ds(b * PT, PT)], isem)
            for b in range(B)
        ]
        pos_cp(0, 0).start()
        pos_cp(1, 1).start()
        for cp in idx_cps:
            cp.wait()
        for b in range(B):
            gather_cp(0, 0, b).start()

        run_group(0, 0, first=True)
        run_group(1, 1, first=True)

        def loop_body(i, _):
            g = 2 + i * NR
            run_group(g, 2 % NR)
            run_group(g + 1, 0)
            run_group(g + 2, 1)
            return 0

        n_steady = (NG - 4) // NR
        lax.fori_loop(0, n_steady, loop_body, 0)

        run_group(NG - 2, (NG - 2) % NR, prefetch=False)
        run_group(NG - 1, (NG - 1) % NR, prefetch=False, last=True)
        for b in range(B):
            out_cp(NG - 2, (NG - 2) % NR, b).wait()
        for b in range(B):
            out_cp(NG - 1, (NG - 1) % NR, b).wait()

    return k


def kernel(idx, token_emb, pos_emb):
    B, T = idx.shape
    V, D = token_emb.shape
    out = _build(B, T, V, D)(idx.astype(jnp.int32), token_emb, pos_emb)
    return out.reshape(B, T, D)


# speedup vs baseline: 1.0174x; 1.0174x over previous
import functools

import jax
import jax.numpy as jnp
from jax import lax
from jax.experimental import pallas as pl
from jax.experimental.pallas import tpu as pltpu
from jax.experimental.pallas import tpu_sc as plsc

_NC, _NS = 2, 16
_NW = _NC * _NS
_LANE = 16
_C = 8
_NR = 3


def _build(B, T, V, D):
    C = _C
    NR = _NR
    PT = T // _NW
    NG = PT // C
    lanes_per_row = D // _LANE

    mesh = plsc.VectorSubcoreMesh(
        core_axis_name="c", subcore_axis_name="s",
        num_cores=_NC, num_subcores=_NS)

    @functools.partial(
        pl.kernel,
        mesh=mesh,
        out_type=jax.ShapeDtypeStruct((B * T, D), jnp.float32),
        scratch_types=[
            pltpu.VMEM((B * PT,), jnp.int32),
            pltpu.VMEM((NR, B, C, D), jnp.float32),
            pltpu.VMEM((NR, C, D), jnp.float32),
            pltpu.SemaphoreType.DMA,
            [[pltpu.SemaphoreType.DMA] * B] * _NR,
            [[pltpu.SemaphoreType.DMA] * B] * _NR,
            [pltpu.SemaphoreType.DMA] * _NR,
        ],
    )
    def k(idx_hbm, tok_hbm, pos_hbm, out_hbm, idx_v, tok_v, pos_v,
          isem, gsems, osems, psems):
        wid = lax.axis_index("s") * _NC + lax.axis_index("c")
        t0 = wid * PT

        def pos_cp(g, r):
            return pltpu.make_async_copy(
                pos_hbm.at[pl.ds(t0 + g * C, C)], pos_v.at[r], psems[r])

        def gather_cp(g, r, b):
            return pltpu.make_async_copy(
                tok_hbm.at[idx_v.at[pl.ds(b * PT + g * C, C)]],
                tok_v.at[r, b], gsems[r][b])

        def out_cp(g, r, b):
            return pltpu.make_async_copy(
                tok_v.at[r, b], out_hbm.at[pl.ds(b * T + t0 + g * C, C)],
                osems[r][b])

        def fused_add(r):
            def row_body(row, _):
                for j in range(lanes_per_row):
                    s = pl.ds(j * _LANE, _LANE)
                    p = pos_v[r, row, s]
                    for b in range(B):
                        plsc.addupdate(tok_v.at[r, b, row, s], p)
                return 0
            lax.fori_loop(0, C, row_body, 0)

        def run_group(g, gr, first=False, prefetch=True, last=False):
            nr = (gr + 1) % NR
            if not first:
                for b in range(B):
                    out_cp(g - 2, nr, b).wait()
            if not last:
                for b in range(B):
                    gather_cp(g + 1, nr, b).start()
            pos_cp(g, gr).wait()
            if prefetch:
                pos_cp(g + 2, (gr + 2) % NR).start()
            for b in range(B):
                gather_cp(g, gr, b).wait()
            fused_add(gr)
            for b in range(B):
                out_cp(g, gr, b).start()

        idx_cps = [
            pltpu.async_copy(idx_hbm.at[b, pl.ds(t0, PT)],
                             idx_v.at[pl.ds(b * PT, PT)], isem)
            for b in range(B)
        ]
        pos_cp(0, 0).start()
        pos_cp(1, 1).start()
        for cp in idx_cps:
            cp.wait()
        for b in range(B):
            gather_cp(0, 0, b).start()

        run_group(0, 0, first=True)
        run_group(1, 1, first=True)

        def loop_body(i, _):
            g = 2 + i * NR
            run_group(g, 2 % NR)
            run_group(g + 1, 0)
            run_group(g + 2, 1)
            return 0

        n_steady = (NG - 4) // NR
        lax.fori_loop(0, n_steady, loop_body, 0)

        run_group(NG - 2, (NG - 2) % NR, prefetch=False)
        run_group(NG - 1, (NG - 1) % NR, prefetch=False, last=True)
        for b in range(B):
            out_cp(NG - 2, (NG - 2) % NR, b).wait()
        for b in range(B):
            out_cp(NG - 1, (NG - 1) % NR, b).wait()

    return k


def kernel(idx, token_emb, pos_emb):
    B, T = idx.shape
    V, D = token_emb.shape
    out = _build(B, T, V, D)(idx.astype(jnp.int32), token_emb, pos_emb)
    return out.reshape(B, T, D)
